# baseline (speedup 1.0000x reference)
import functools

import jax
import jax.numpy as jnp
from jax import lax
from jax.experimental import pallas as pl
from jax.experimental.pallas import tpu as pltpu
from jax.experimental.pallas import tpu_sc as plsc

B = 16384
EMB = 64
HID = 192
EPS = 1e-5
NC, NS = 2, 16
NW = NC * NS
BPW = B // NW
CG = 128


def _sc_gather(idx_all, t1, t2, t3, t4):
    mesh = plsc.VectorSubcoreMesh(core_axis_name="c", subcore_axis_name="s")
    out_type = jax.ShapeDtypeStruct((4 * B, 2 * EMB), jnp.float32)
    scratch = (
        [pltpu.VMEM((BPW,), jnp.int32)]
        + [pltpu.VMEM((BPW, 2 * EMB), jnp.float32)]
        + [pltpu.SemaphoreType.DMA]
    )

    @functools.partial(pl.kernel, mesh=mesh, out_type=out_type,
                       scratch_types=scratch)
    def k(idx_r, t1r, t2r, t3r, t4r, out_r, tidx_v, rows_v, sem):
        wid = lax.axis_index("s") * NC + lax.axis_index("c")
        base = wid * BPW

        def gather_one(t, tab):
            pltpu.sync_copy(idx_r.at[pl.ds(t * B + base, BPW)], tidx_v)

            def halve(g, _):
                v = tidx_v[pl.ds(g * 16, 16)]
                tidx_v[pl.ds(g * 16, 16)] = lax.shift_right_logical(v, 1)
                return 0
            lax.fori_loop(0, BPW // 16, halve, 0)
            for c in range(BPW // CG):
                pltpu.async_copy(
                    tab.at[tidx_v.at[pl.ds(c * CG, CG)]],
                    rows_v.at[pl.ds(c * CG, CG), :], sem)
            pltpu.make_async_copy(out_r.at[pl.ds(0, BPW), :], rows_v,
                                  sem).wait()
            pltpu.sync_copy(rows_v, out_r.at[pl.ds(t * B + base, BPW), :])

        gather_one(0, t1r)
        gather_one(1, t2r)
        gather_one(2, t3r)
        gather_one(3, t4r)

    return k(idx_all, t1, t2, t3, t4)


BCHUNK = 1024
NBCHUNK = B // BCHUNK
CAT = 4 * EMB


def _select(e, p):
    left = e[:, :EMB]
    right = e[:, EMB:]
    return left + p * (right - left)


def _stats_body(e1, e2, e3, e4, p1, p2, p3, p4, gamma, beta, g, v, bias,
                ws_out, b2_out, acc):
    step = pl.program_id(0)

    @pl.when(step == 0)
    def _init():
        acc[...] = jnp.zeros_like(acc)

    x = jnp.concatenate(
        [_select(e[...], p[...])
         for e, p in ((e1, p1), (e2, p2), (e3, p3), (e4, p4))], axis=1)
    acc[0:1, :] += jnp.sum(x, axis=0, keepdims=True)
    acc[1:2, :] += jnp.sum(x * x, axis=0, keepdims=True)

    @pl.when(step == NBCHUNK - 1)
    def _finalize():
        mean = acc[0:1, :] / B
        var = acc[1:2, :] / B - mean * mean
        s = gamma[...][None, :] / jnp.sqrt(var + EPS)
        shift = beta[...][None, :] - mean * s
        vv = v[...]
        v_norm = jnp.sqrt(jnp.sum(vv * vv, axis=1, keepdims=True))
        W = (g[...][:, None] / v_norm) * vv
        ws_out[...] = W * s
        b2 = bias[...] + lax.dot_general(
            W, shift[0], (((1,), (0,)), ((), ())),
            preferred_element_type=jnp.float32)
        b2_out[...] = b2[None, :]


def _matmul_body(e1, e2, e3, e4, p1, p2, p3, p4, ws, b2, out):
    x = jnp.concatenate(
        [_select(e[...], p[...])
         for e, p in ((e1, p1), (e2, p2), (e3, p3), (e4, p4))], axis=1)
    y = lax.dot_general(x, ws[...], (((1,), (1,)), ((), ())),
                        preferred_element_type=jnp.float32)
    out[...] = jax.nn.sigmoid(y + b2[...])


def _tc_stage(e_all, par_all, bn_gamma, bn_beta, wn_g, wn_v, bias):
    def espec(t, width):
        return pl.BlockSpec((BCHUNK, width),
                            functools.partial(
                                lambda tt, i: (tt * NBCHUNK + i, 0), t))
    especs = [espec(t, 2 * EMB) for t in range(4)]
    pspecs = [espec(t, EMB) for t in range(4)]
    full = lambda shape: pl.BlockSpec(shape, lambda i: tuple(0 for _ in shape))
    earr = [e_all] * 4 + [par_all] * 4
    ws, b2 = pl.pallas_call(
        _stats_body,
        grid=(NBCHUNK,),
        in_specs=especs + pspecs + [full((CAT,)), full((CAT,)), full((HID,)),
                                    full((HID, CAT)), full((HID,))],
        out_specs=[full((HID, CAT)), full((1, HID))],
        out_shape=[jax.ShapeDtypeStruct((HID, CAT), jnp.float32),
                   jax.ShapeDtypeStruct((1, HID), jnp.float32)],
        scratch_shapes=[pltpu.VMEM((2, CAT), jnp.float32)],
    )(*earr, bn_gamma, bn_beta, wn_g, wn_v, bias)
    out = pl.pallas_call(
        _matmul_body,
        grid=(NBCHUNK,),
        in_specs=especs + pspecs + [full((HID, CAT)), full((1, HID))],
        out_specs=pl.BlockSpec((BCHUNK, HID), lambda i: (i, 0)),
        out_shape=jax.ShapeDtypeStruct((B, HID), jnp.float32),
    )(*earr, ws, b2)
    return out


def kernel(last_test, last_question, last_tag, last_qclass,
           emb_test, emb_question, emb_tag, emb_qclass,
           bn_gamma, bn_beta, wn_g, wn_v, bias):
    idx_all = jnp.concatenate([
        last_test.astype(jnp.int32), last_question.astype(jnp.int32),
        last_tag.astype(jnp.int32), last_qclass.astype(jnp.int32)])
    par_all = jnp.broadcast_to(
        jnp.bitwise_and(idx_all, 1).astype(jnp.float32)[:, None],
        (4 * B, EMB))
    e_all = _sc_gather(idx_all,
                       emb_test.reshape(-1, 2 * EMB),
                       emb_question.reshape(-1, 2 * EMB),
                       emb_tag.reshape(-1, 2 * EMB),
                       emb_qclass.reshape(-1, 2 * EMB))
    return _tc_stage(e_all, par_all, bn_gamma, bn_beta, wn_g, wn_v, bias)


# speedup vs baseline: 2.4067x; 2.4067x over previous
import functools

import jax
import jax.numpy as jnp
from jax import lax
from jax.experimental import pallas as pl
from jax.experimental.pallas import tpu as pltpu
from jax.experimental.pallas import tpu_sc as plsc

B = 16384
EMB = 64
HID = 192
EPS = 1e-5
NC, NS = 2, 16
NW = NC * NS
BPW = B // NW


def _sc_gather(idx_all, t1, t2, t3, t4):
    mesh = plsc.VectorSubcoreMesh(core_axis_name="c", subcore_axis_name="s")
    out_type = jax.ShapeDtypeStruct((4 * B, EMB), jnp.float32)
    scratch = (
        [pltpu.VMEM((BPW,), jnp.int32)]
        + [pltpu.VMEM((BPW, EMB), jnp.float32)]
        + [pltpu.SemaphoreType.DMA]
    )

    @functools.partial(pl.kernel, mesh=mesh, out_type=out_type,
                       scratch_types=scratch)
    def k(idx_r, t1r, t2r, t3r, t4r, out_r, idx_v, rows_v, sem):
        wid = lax.axis_index("s") * NC + lax.axis_index("c")
        base = wid * BPW

        def gather_one(t, tab):
            pltpu.sync_copy(idx_r.at[pl.ds(t * B + base, BPW)], idx_v)

            def grp(g, _):
                vec = idx_v[pl.ds(g * 16, 16)]
                tid = lax.shift_right_logical(vec, 3)
                ph = jnp.bitwise_and(vec, 7)
                for j in range(16):
                    pltpu.async_copy(
                        tab.at[tid[j], pl.ds(ph[j], 1), :],
                        rows_v.at[pl.ds(g * 16 + j, 1), :], sem)
                return 0
            lax.fori_loop(0, BPW // 16, grp, 0)
            pltpu.make_async_copy(out_r.at[pl.ds(0, BPW), :], rows_v,
                                  sem).wait()
            pltpu.sync_copy(rows_v, out_r.at[pl.ds(t * B + base, BPW), :])

        gather_one(0, t1r)
        gather_one(1, t2r)
        gather_one(2, t3r)
        gather_one(3, t4r)

    return k(idx_all, t1, t2, t3, t4)


BCHUNK = 1024
NBCHUNK = B // BCHUNK
CAT = 4 * EMB


def _dense_body(e1, e2, e3, e4, gamma, beta, g, v, bias, out,
                acc, ws_s, b2_s, x_cache):
    phase = pl.program_id(0)
    step = pl.program_id(1)

    @pl.when(phase == 0)
    def _stats():
        @pl.when(step == 0)
        def _init():
            acc[...] = jnp.zeros_like(acc)

        x = jnp.concatenate([e1[...], e2[...], e3[...], e4[...]], axis=1)
        x_cache[pl.ds(step * BCHUNK, BCHUNK), :] = x
        acc[0:1, :] += jnp.sum(x, axis=0, keepdims=True)
        acc[1:2, :] += jnp.sum(x * x, axis=0, keepdims=True)

        @pl.when(step == NBCHUNK - 1)
        def _finalize():
            mean = acc[0:1, :] / B
            var = acc[1:2, :] / B - mean * mean
            s = gamma[...][None, :] / jnp.sqrt(var + EPS)
            shift = beta[...][None, :] - mean * s
            vv = v[...]
            v_norm = jnp.sqrt(jnp.sum(vv * vv, axis=1, keepdims=True))
            W = (g[...][:, None] / v_norm) * vv
            ws_s[...] = W * s
            b2 = bias[...] + lax.dot_general(
                W, shift[0], (((1,), (0,)), ((), ())),
                preferred_element_type=jnp.float32)
            b2_s[...] = b2[None, :]

    @pl.when(phase == 1)
    def _matmul():
        x = x_cache[pl.ds(step * BCHUNK, BCHUNK), :]
        y = lax.dot_general(x, ws_s[...], (((1,), (1,)), ((), ())),
                            preferred_element_type=jnp.float32)
        out[...] = jax.nn.sigmoid(y + b2_s[...])


def _tc_stage(e_all, bn_gamma, bn_beta, wn_g, wn_v, bias):
    especs = [
        pl.BlockSpec(
            (BCHUNK, EMB),
            functools.partial(
                lambda t, p, i: (t * NBCHUNK + jnp.where(p == 0, i, 0), 0), t))
        for t in range(4)]
    full = lambda shape: pl.BlockSpec(
        shape, lambda p, i: tuple(0 for _ in shape))
    out = pl.pallas_call(
        _dense_body,
        grid=(2, NBCHUNK),
        in_specs=especs + [full((CAT,)), full((CAT,)), full((HID,)),
                           full((HID, CAT)), full((HID,))],
        out_specs=pl.BlockSpec((BCHUNK, HID),
                               lambda p, i: (jnp.where(p == 0, 0, i), 0)),
        out_shape=jax.ShapeDtypeStruct((B, HID), jnp.float32),
        scratch_shapes=[pltpu.VMEM((2, CAT), jnp.float32),
                        pltpu.VMEM((HID, CAT), jnp.float32),
                        pltpu.VMEM((1, HID), jnp.float32),
                        pltpu.VMEM((B, CAT), jnp.float32)],
    )(e_all, e_all, e_all, e_all, bn_gamma, bn_beta, wn_g, wn_v, bias)
    return out


def kernel(last_test, last_question, last_tag, last_qclass,
           emb_test, emb_question, emb_tag, emb_qclass,
           bn_gamma, bn_beta, wn_g, wn_v, bias):
    idx_all = jnp.concatenate([
        last_test.astype(jnp.int32), last_question.astype(jnp.int32),
        last_tag.astype(jnp.int32), last_qclass.astype(jnp.int32)])
    e_all = _sc_gather(idx_all,
                       emb_test.reshape(-1, 8, EMB),
                       emb_question.reshape(-1, 8, EMB),
                       emb_tag.reshape(-1, 8, EMB),
                       emb_qclass.reshape(-1, 8, EMB))
    return _tc_stage(e_all, bn_gamma, bn_beta, wn_g, wn_v, bias)
